# baseline (speedup 1.0000x reference)
import functools

import jax
import jax.numpy as jnp
from jax import lax
from jax.experimental import pallas as pl
from jax.experimental.pallas import tpu as pltpu
from jax.experimental.pallas import tpu_sc as plsc

N_NODES = 10000
NPAD = 10240
E = 320000
N_SUB = 512
H = 128
AW = 8

HH = H // 2

CH = 80
NCHUNK = E // CH
CPH = NCHUNK // 2 // 16
TILE_ROWS = NPAD // 16

R = 1000
NB = N_NODES // R


def _sc_segment_sums(xs, src, dst, ea16, zg, za):
    mesh = plsc.VectorSubcoreMesh(core_axis_name="c", subcore_axis_name="s")

    @functools.partial(
        pl.kernel,
        out_type=[
            jax.ShapeDtypeStruct((2 * NPAD, HH), jnp.float32),
            jax.ShapeDtypeStruct((2 * NPAD, AW), jnp.float32),
        ],
        mesh=mesh,
        compiler_params=pltpu.CompilerParams(use_tc_tiling_on_sc=False),
        scratch_types=[
            pltpu.VMEM((CPH, CH), jnp.int32),
            pltpu.VMEM((CPH, CH), jnp.int32),
            pltpu.VMEM((CH, HH), jnp.float32),
            pltpu.VMEM((CH, HH), jnp.float32),
            pltpu.VMEM((CH, AW), jnp.float32),
            pltpu.VMEM((CH, AW), jnp.float32),
            pltpu.VMEM((TILE_ROWS // 4, HH), jnp.float32),
            pltpu.VMEM((TILE_ROWS // 4, AW), jnp.float32),
            pltpu.VMEM_SHARED((NPAD, HH), jnp.float32),
            pltpu.VMEM_SHARED((NPAD, AW), jnp.float32),
            pltpu.SemaphoreType.DMA,
            pltpu.SemaphoreType.DMA,
        ],
    )
    def sc_kernel(xs_hbm, src_hbm, dst_hbm, ea_hbm, zg_hbm, za_hbm,
                  g_out, a_out, srcblk, dstblk, rowb0, rowb1,
                  eab0, eab1, stg_g, stg_a, gacc, aacc, sem0, sem1):
        cid = lax.axis_index("c")
        sid = lax.axis_index("s")
        r0 = sid * TILE_ROWS
        rb = (rowb0, rowb1)
        eb = (eab0, eab1)
        sems = (sem0, sem1)

        QR = TILE_ROWS // 4
        for q in range(4):
            pltpu.sync_copy(zg_hbm.at[pl.ds(r0 + q * QR, QR)], stg_g)
            pltpu.sync_copy(stg_g, gacc.at[pl.ds(r0 + q * QR, QR)])
            pltpu.sync_copy(za_hbm.at[pl.ds(r0 + q * QR, QR)], stg_a)
            pltpu.sync_copy(stg_a, aacc.at[pl.ds(r0 + q * QR, QR)])
        plsc.subcore_barrier()

        own0 = cid * (NCHUNK // 2) + sid * CPH
        oth0 = (1 - cid) * (NCHUNK // 2) + sid * CPH

        def run_half(c0, own):
            pltpu.sync_copy(src_hbm.at[pl.ds(c0, CPH)], srcblk)
            pltpu.sync_copy(dst_hbm.at[pl.ds(c0, CPH)], dstblk)
            xh = xs_hbm.at[cid]

            def fire(i, p):
                if own:
                    pltpu.sync_copy(ea_hbm.at[c0 + i], eb[p])
                pltpu.async_copy(xh.at[srcblk.at[i]], rb[p], sems[p])

            def drain(i, p):
                pltpu.make_async_copy(
                    xh.at[srcblk.at[i]], rb[p], sems[p]).wait()
                pltpu.sync_copy(rb[p], gacc.at[dstblk.at[i]], add=True)
                if own:
                    pltpu.sync_copy(eb[p], aacc.at[dstblk.at[i]], add=True)

            fire(0, 0)

            def pair(k, carry):
                i = 2 * k
                fire(i + 1, 1)
                drain(i, 0)
                fire(i + 2, 0)
                drain(i + 1, 1)
                return carry

            lax.fori_loop(0, (CPH - 1) // 2, pair, 0)
            drain(CPH - 1, 0)

        run_half(own0, True)
        run_half(oth0, False)
        plsc.subcore_barrier()

        for q in range(4):
            pltpu.sync_copy(gacc.at[pl.ds(r0 + q * QR, QR)], stg_g)
            pltpu.sync_copy(
                stg_g, g_out.at[pl.ds(cid * NPAD + r0 + q * QR, QR)])
            pltpu.sync_copy(aacc.at[pl.ds(r0 + q * QR, QR)], stg_a)
            pltpu.sync_copy(
                stg_a, a_out.at[pl.ds(cid * NPAD + r0 + q * QR, QR)])

    gp, ap = sc_kernel(
        xs,
        src.reshape(NCHUNK, CH),
        dst.reshape(NCHUNK, CH),
        ea16.reshape(NCHUNK, CH, AW),
        zg, za)
    return gp.reshape(2, NPAD, HH), ap.reshape(2, NPAD, AW)


def _tc_xw_body(x_ref, wn_ref, t_ref):
    t_ref[...] = jnp.dot(x_ref[...], wn_ref[...],
                         preferred_element_type=jnp.float32)


def _tc_body(t1_ref, gp_ref, ap_ref, batch_ref, wn_ref, we_ref, wu_ref,
             bn_ref, bu_ref, out_ref, cnt_ref):
    i = pl.program_id(0)

    @pl.when(i == 0)
    def _init():
        out_ref[...] = jnp.zeros_like(out_ref)
        cnt_ref[...] = jnp.zeros_like(cnt_ref)

    a = ap_ref[0] + ap_ref[1]
    wn = wn_ref[...]
    z = (
        t1_ref[...]
        + jnp.dot(gp_ref[0], wn[:HH], preferred_element_type=jnp.float32)
        + jnp.dot(gp_ref[1], wn[HH:], preferred_element_type=jnp.float32)
        + jnp.dot(a, we_ref[...], preferred_element_type=jnp.float32)
        + bn_ref[...]
    )
    y = jnp.maximum(jnp.dot(z, wu_ref[...], preferred_element_type=jnp.float32)
                    + bu_ref[...], 0.0)

    seg = batch_ref[0]
    onehot = (seg == lax.broadcasted_iota(jnp.int32, (N_SUB, R), 0)
              ).astype(jnp.float32)
    out_ref[...] += jnp.dot(onehot, y, preferred_element_type=jnp.float32)
    cnt_ref[...] += jnp.sum(onehot, axis=1, keepdims=True)

    @pl.when(i == NB - 1)
    def _fin():
        out_ref[...] = out_ref[...] / jnp.maximum(cnt_ref[...], 1.0)


def kernel(x, edge_attr, W_node, b_node, W_edge, b_edge, W_upd, b_upd,
           batch, subgraph_idx_batch, edge_index):
    src = edge_index[0]
    dst = edge_index[1]
    ea16 = jnp.concatenate(
        [edge_attr,
         jnp.ones((E, 1), jnp.float32),
         jnp.zeros((E, AW - 4), jnp.float32)], axis=1)
    we16 = jnp.concatenate(
        [W_edge,
         (b_node + b_edge)[None, :],
         jnp.zeros((AW - 4, H), jnp.float32)], axis=0)
    zg = jnp.zeros((NPAD, HH), jnp.float32)
    za = jnp.zeros((NPAD, AW), jnp.float32)
    xs = jnp.stack([x[:, :HH], x[:, HH:]])

    gp, ap = _sc_segment_sums(xs, src, dst, ea16, zg, za)

    t1 = pl.pallas_call(
        _tc_xw_body,
        grid=(NB,),
        in_specs=[
            pl.BlockSpec((R, H), lambda i: (i, 0)),
            pl.BlockSpec((H, H), lambda i: (0, 0)),
        ],
        out_specs=pl.BlockSpec((R, H), lambda i: (i, 0)),
        out_shape=jax.ShapeDtypeStruct((N_NODES, H), jnp.float32),
    )(x, W_node)

    out = pl.pallas_call(
        _tc_body,
        grid=(NB,),
        in_specs=[
            pl.BlockSpec((R, H), lambda i: (i, 0)),
            pl.BlockSpec((2, R, HH), lambda i: (0, i, 0)),
            pl.BlockSpec((2, R, AW), lambda i: (0, i, 0)),
            pl.BlockSpec((1, 1, R), lambda i: (i, 0, 0)),
            pl.BlockSpec((H, H), lambda i: (0, 0)),
            pl.BlockSpec((AW, H), lambda i: (0, 0)),
            pl.BlockSpec((H, H), lambda i: (0, 0)),
            pl.BlockSpec((1, H), lambda i: (0, 0)),
            pl.BlockSpec((1, H), lambda i: (0, 0)),
        ],
        out_specs=pl.BlockSpec((N_SUB, H), lambda i: (0, 0)),
        out_shape=jax.ShapeDtypeStruct((N_SUB, H), jnp.float32),
        scratch_shapes=[pltpu.VMEM((N_SUB, H), jnp.float32)],
    )(t1, gp, ap, batch.reshape(NB, 1, R), W_node, we16, W_upd,
      b_node[None, :], b_upd[None, :])
    return out


# speedup vs baseline: 1.4878x; 1.4878x over previous
import functools

import jax
import jax.numpy as jnp
from jax import lax
from jax.experimental import pallas as pl
from jax.experimental.pallas import tpu as pltpu
from jax.experimental.pallas import tpu_sc as plsc

N_NODES = 10000
NPAD = 10240
E = 320000
N_SUB = 512
H = 128
AW = 4

HH = H // 2

CH = 80
NCHUNK = E // CH
CPH = NCHUNK // 2 // 16
TILE_ROWS = NPAD // 16

R = 1000
NB = N_NODES // R


def _sc_segment_sums(xs, src, dst, e0, e1, e2, ones_e, zg, za):
    mesh = plsc.VectorSubcoreMesh(core_axis_name="c", subcore_axis_name="s")
    HCH = CPH * CH

    @functools.partial(
        pl.kernel,
        out_type=[
            jax.ShapeDtypeStruct((2 * NPAD, HH), jnp.float32),
            jax.ShapeDtypeStruct((2, AW, NPAD), jnp.float32),
        ],
        mesh=mesh,
        compiler_params=pltpu.CompilerParams(use_tc_tiling_on_sc=False),
        scratch_types=[
            pltpu.VMEM((CPH, CH), jnp.int32),
            pltpu.VMEM((CPH, CH), jnp.int32),
            pltpu.VMEM((CH, HH), jnp.float32),
            pltpu.VMEM((CH, HH), jnp.float32),
            pltpu.VMEM((HCH,), jnp.float32),
            pltpu.VMEM((HCH,), jnp.float32),
            pltpu.VMEM((HCH,), jnp.float32),
            pltpu.VMEM((CH,), jnp.float32),
            pltpu.VMEM((TILE_ROWS // 4, HH), jnp.float32),
            pltpu.VMEM((TILE_ROWS,), jnp.float32),
            pltpu.VMEM_SHARED((NPAD, HH), jnp.float32),
            pltpu.VMEM_SHARED((AW, NPAD), jnp.float32),
            pltpu.SemaphoreType.DMA,
            pltpu.SemaphoreType.DMA,
        ],
    )
    def sc_kernel(xs_hbm, src_hbm, dst_hbm, e0_hbm, e1_hbm, e2_hbm,
                  ones_hbm, zg_hbm, za_hbm, g_out, a_out,
                  srcblk, dstblk, rowb0, rowb1, ec0, ec1, ec2, onesb,
                  stg_g, stg_a, gacc, aacc, sem0, sem1):
        cid = lax.axis_index("c")
        sid = lax.axis_index("s")
        r0 = sid * TILE_ROWS
        rb = (rowb0, rowb1)
        sems = (sem0, sem1)

        QR = TILE_ROWS // 4
        for q in range(4):
            pltpu.sync_copy(zg_hbm.at[pl.ds(r0 + q * QR, QR)], stg_g)
            pltpu.sync_copy(stg_g, gacc.at[pl.ds(r0 + q * QR, QR)])
        pltpu.sync_copy(za_hbm.at[pl.ds(r0, TILE_ROWS)], stg_a)
        for c in range(AW):
            pltpu.sync_copy(stg_a, aacc.at[c].at[pl.ds(r0, TILE_ROWS)])
        pltpu.sync_copy(ones_hbm, onesb)
        plsc.subcore_barrier()

        own0 = cid * (NCHUNK // 2) + sid * CPH
        oth0 = (1 - cid) * (NCHUNK // 2) + sid * CPH

        def run_half(c0, own):
            pltpu.sync_copy(src_hbm.at[pl.ds(c0, CPH)], srcblk)
            pltpu.sync_copy(dst_hbm.at[pl.ds(c0, CPH)], dstblk)
            if own:
                pltpu.sync_copy(e0_hbm.at[pl.ds(c0 * CH, HCH)], ec0)
                pltpu.sync_copy(e1_hbm.at[pl.ds(c0 * CH, HCH)], ec1)
                pltpu.sync_copy(e2_hbm.at[pl.ds(c0 * CH, HCH)], ec2)
            xh = xs_hbm.at[cid]

            def fire(i, p):
                pltpu.async_copy(xh.at[srcblk.at[i]], rb[p], sems[p])

            def drain(i, p):
                pltpu.make_async_copy(
                    xh.at[srcblk.at[i]], rb[p], sems[p]).wait()
                pltpu.sync_copy(rb[p], gacc.at[dstblk.at[i]], add=True)
                if own:
                    dsts = dstblk.at[i]
                    pltpu.sync_copy(ec0.at[pl.ds(i * CH, CH)],
                                    aacc.at[0].at[dsts], add=True)
                    pltpu.sync_copy(ec1.at[pl.ds(i * CH, CH)],
                                    aacc.at[1].at[dsts], add=True)
                    pltpu.sync_copy(ec2.at[pl.ds(i * CH, CH)],
                                    aacc.at[2].at[dsts], add=True)
                    pltpu.sync_copy(onesb, aacc.at[3].at[dsts], add=True)

            fire(0, 0)

            def pair(k, carry):
                i = 2 * k
                fire(i + 1, 1)
                drain(i, 0)
                fire(i + 2, 0)
                drain(i + 1, 1)
                return carry

            lax.fori_loop(0, (CPH - 1) // 2, pair, 0)
            drain(CPH - 1, 0)

        run_half(own0, True)
        run_half(oth0, False)
        plsc.subcore_barrier()

        for q in range(4):
            pltpu.sync_copy(gacc.at[pl.ds(r0 + q * QR, QR)], stg_g)
            pltpu.sync_copy(
                stg_g, g_out.at[pl.ds(cid * NPAD + r0 + q * QR, QR)])
        for c in range(AW):
            pltpu.sync_copy(aacc.at[c].at[pl.ds(r0, TILE_ROWS)], stg_a)
            pltpu.sync_copy(
                stg_a, a_out.at[cid].at[c].at[pl.ds(r0, TILE_ROWS)])

    gp, ap = sc_kernel(
        xs,
        src.reshape(NCHUNK, CH),
        dst.reshape(NCHUNK, CH),
        e0, e1, e2, ones_e, zg, za)
    return gp.reshape(2, NPAD, HH), ap


def _tc_xw_body(x_ref, wn_ref, t_ref):
    t_ref[...] = jnp.dot(x_ref[...], wn_ref[...],
                         preferred_element_type=jnp.float32)


def _tc_body(t1_ref, gp_ref, ap_ref, batch_ref, wn_ref, we_ref, wu_ref,
             bn_ref, bu_ref, out_ref, cnt_ref):
    i = pl.program_id(0)

    @pl.when(i == 0)
    def _init():
        out_ref[...] = jnp.zeros_like(out_ref)
        cnt_ref[...] = jnp.zeros_like(cnt_ref)

    a = ap_ref[0] + ap_ref[1]
    wn = wn_ref[...]
    z = (
        t1_ref[...]
        + jnp.dot(gp_ref[0], wn[:HH], preferred_element_type=jnp.float32)
        + jnp.dot(gp_ref[1], wn[HH:], preferred_element_type=jnp.float32)
        + jnp.dot(a, we_ref[...], preferred_element_type=jnp.float32)
        + bn_ref[...]
    )
    y = jnp.maximum(jnp.dot(z, wu_ref[...], preferred_element_type=jnp.float32)
                    + bu_ref[...], 0.0)

    seg = batch_ref[0]
    onehot = (seg == lax.broadcasted_iota(jnp.int32, (N_SUB, R), 0)
              ).astype(jnp.float32)
    out_ref[...] += jnp.dot(onehot, y, preferred_element_type=jnp.float32)
    cnt_ref[...] += jnp.sum(onehot, axis=1, keepdims=True)

    @pl.when(i == NB - 1)
    def _fin():
        out_ref[...] = out_ref[...] / jnp.maximum(cnt_ref[...], 1.0)


def kernel(x, edge_attr, W_node, b_node, W_edge, b_edge, W_upd, b_upd,
           batch, subgraph_idx_batch, edge_index):
    src = edge_index[0]
    dst = edge_index[1]
    we4 = jnp.concatenate([W_edge, (b_node + b_edge)[None, :]], axis=0)
    zg = jnp.zeros((NPAD, HH), jnp.float32)
    za = jnp.zeros((NPAD,), jnp.float32)
    ones_e = jnp.ones((CH,), jnp.float32)
    xs = jnp.stack([x[:, :HH], x[:, HH:]])

    gp, ap = _sc_segment_sums(
        xs, src, dst,
        edge_attr[:, 0], edge_attr[:, 1], edge_attr[:, 2],
        ones_e, zg, za)

    t1 = pl.pallas_call(
        _tc_xw_body,
        grid=(NB,),
        in_specs=[
            pl.BlockSpec((R, H), lambda i: (i, 0)),
            pl.BlockSpec((H, H), lambda i: (0, 0)),
        ],
        out_specs=pl.BlockSpec((R, H), lambda i: (i, 0)),
        out_shape=jax.ShapeDtypeStruct((N_NODES, H), jnp.float32),
    )(x, W_node)

    out = pl.pallas_call(
        _tc_body,
        grid=(NB,),
        in_specs=[
            pl.BlockSpec((R, H), lambda i: (i, 0)),
            pl.BlockSpec((2, R, HH), lambda i: (0, i, 0)),
            pl.BlockSpec((2, R, AW), lambda i: (0, i, 0)),
            pl.BlockSpec((1, 1, R), lambda i: (i, 0, 0)),
            pl.BlockSpec((H, H), lambda i: (0, 0)),
            pl.BlockSpec((AW, H), lambda i: (0, 0)),
            pl.BlockSpec((H, H), lambda i: (0, 0)),
            pl.BlockSpec((1, H), lambda i: (0, 0)),
            pl.BlockSpec((1, H), lambda i: (0, 0)),
        ],
        out_specs=pl.BlockSpec((N_SUB, H), lambda i: (0, 0)),
        out_shape=jax.ShapeDtypeStruct((N_SUB, H), jnp.float32),
        scratch_shapes=[pltpu.VMEM((N_SUB, H), jnp.float32)],
    )(t1, gp, jnp.swapaxes(ap, 1, 2), batch.reshape(NB, 1, R), W_node,
      we4, W_upd,
      b_node[None, :], b_upd[None, :])
    return out


# speedup vs baseline: 1.5746x; 1.0583x over previous
import functools

import jax
import jax.numpy as jnp
from jax import lax
from jax.experimental import pallas as pl
from jax.experimental.pallas import tpu as pltpu
from jax.experimental.pallas import tpu_sc as plsc

N_NODES = 10000
NPAD = 10240
E = 320000
N_SUB = 512
H = 128
AW = 4

HH = H // 2

CH = 80
NCHUNK = E // CH
CPH = NCHUNK // 2 // 16
TILE_ROWS = NPAD // 16

R = 1000
NB = N_NODES // R


def _sc_segment_sums(xs, src, dst, e0, e1, e2, ones_e, zg, za):
    mesh = plsc.VectorSubcoreMesh(core_axis_name="c", subcore_axis_name="s")
    HCH = CPH * CH

    @functools.partial(
        pl.kernel,
        out_type=[
            jax.ShapeDtypeStruct((2 * NPAD, HH), jnp.float32),
            jax.ShapeDtypeStruct((2, AW, NPAD), jnp.float32),
        ],
        mesh=mesh,
        compiler_params=pltpu.CompilerParams(use_tc_tiling_on_sc=False),
        scratch_types=[
            pltpu.VMEM((CPH, CH), jnp.int32),
            pltpu.VMEM((CPH, CH), jnp.int32),
            pltpu.VMEM((CH, HH), jnp.float32),
            pltpu.VMEM((CH, HH), jnp.float32),
            pltpu.VMEM((HCH,), jnp.float32),
            pltpu.VMEM((HCH,), jnp.float32),
            pltpu.VMEM((HCH,), jnp.float32),
            pltpu.VMEM((CH,), jnp.float32),
            pltpu.VMEM((TILE_ROWS // 4, HH), jnp.float32),
            pltpu.VMEM((TILE_ROWS,), jnp.float32),
            pltpu.VMEM_SHARED((NPAD, HH), jnp.float32),
            pltpu.VMEM_SHARED((AW, NPAD), jnp.float32),
            pltpu.SemaphoreType.DMA,
            pltpu.SemaphoreType.DMA,
            pltpu.SemaphoreType.DMA,
        ],
    )
    def sc_kernel(xs_hbm, src_hbm, dst_hbm, e0_hbm, e1_hbm, e2_hbm,
                  ones_hbm, zg_hbm, za_hbm, g_out, a_out,
                  srcblk, dstblk, rowb0, rowb1, ec0, ec1, ec2, onesb,
                  stg_g, stg_a, gacc, aacc, sem0, sem1, sem2):
        cid = lax.axis_index("c")
        sid = lax.axis_index("s")
        r0 = sid * TILE_ROWS
        rb = (rowb0, rowb1)
        sems = (sem0, sem1)

        QR = TILE_ROWS // 4
        for q in range(4):
            pltpu.sync_copy(zg_hbm.at[pl.ds(r0 + q * QR, QR)], stg_g)
            pltpu.sync_copy(stg_g, gacc.at[pl.ds(r0 + q * QR, QR)])
        pltpu.sync_copy(za_hbm.at[pl.ds(r0, TILE_ROWS)], stg_a)
        for c in range(AW):
            pltpu.sync_copy(stg_a, aacc.at[c].at[pl.ds(r0, TILE_ROWS)])
        pltpu.sync_copy(ones_hbm, onesb)
        plsc.subcore_barrier()

        own0 = cid * (NCHUNK // 2) + sid * CPH
        oth0 = (1 - cid) * (NCHUNK // 2) + sid * CPH

        def run_half(c0, own):
            pltpu.sync_copy(src_hbm.at[pl.ds(c0, CPH)], srcblk)
            pltpu.sync_copy(dst_hbm.at[pl.ds(c0, CPH)], dstblk)
            if own:
                pltpu.sync_copy(e0_hbm.at[pl.ds(c0 * CH, HCH)], ec0)
                pltpu.sync_copy(e1_hbm.at[pl.ds(c0 * CH, HCH)], ec1)
                pltpu.sync_copy(e2_hbm.at[pl.ds(c0 * CH, HCH)], ec2)
            xh = xs_hbm.at[cid]

            def fire(i, p):
                pltpu.async_copy(xh.at[srcblk.at[i]], rb[p], sems[p])

            def drain(i, p):
                pltpu.make_async_copy(
                    xh.at[srcblk.at[i]], rb[p], sems[p]).wait()
                if own:
                    dsts = dstblk.at[i]
                    cs0 = pltpu.async_copy(ec0.at[pl.ds(i * CH, CH)],
                                           aacc.at[0].at[dsts], sem2,
                                           add=True)
                    cs1 = pltpu.async_copy(ec1.at[pl.ds(i * CH, CH)],
                                           aacc.at[1].at[dsts], sem2,
                                           add=True)
                    cs2 = pltpu.async_copy(ec2.at[pl.ds(i * CH, CH)],
                                           aacc.at[2].at[dsts], sem2,
                                           add=True)
                    cs3 = pltpu.async_copy(onesb, aacc.at[3].at[dsts],
                                           sem2, add=True)
                    pltpu.sync_copy(rb[p], gacc.at[dstblk.at[i]], add=True)
                    cs0.wait()
                    cs1.wait()
                    cs2.wait()
                    cs3.wait()
                else:
                    pltpu.sync_copy(rb[p], gacc.at[dstblk.at[i]], add=True)

            fire(0, 0)

            def pair(k, carry):
                i = 2 * k
                fire(i + 1, 1)
                drain(i, 0)
                fire(i + 2, 0)
                drain(i + 1, 1)
                return carry

            lax.fori_loop(0, (CPH - 1) // 2, pair, 0)
            drain(CPH - 1, 0)

        run_half(own0, True)
        run_half(oth0, False)
        plsc.subcore_barrier()

        for q in range(4):
            pltpu.sync_copy(gacc.at[pl.ds(r0 + q * QR, QR)], stg_g)
            pltpu.sync_copy(
                stg_g, g_out.at[pl.ds(cid * NPAD + r0 + q * QR, QR)])
        for c in range(AW):
            pltpu.sync_copy(aacc.at[c].at[pl.ds(r0, TILE_ROWS)], stg_a)
            pltpu.sync_copy(
                stg_a, a_out.at[cid].at[c].at[pl.ds(r0, TILE_ROWS)])

    gp, ap = sc_kernel(
        xs,
        src.reshape(NCHUNK, CH),
        dst.reshape(NCHUNK, CH),
        e0, e1, e2, ones_e, zg, za)
    return gp.reshape(2, NPAD, HH), ap


def _tc_xw_body(x_ref, wn_ref, t_ref):
    t_ref[...] = jnp.dot(x_ref[...], wn_ref[...],
                         preferred_element_type=jnp.float32)


def _tc_body(t1_ref, gp_ref, ap_ref, batch_ref, wn_ref, we_ref, wu_ref,
             bn_ref, bu_ref, out_ref, cnt_ref):
    i = pl.program_id(0)

    @pl.when(i == 0)
    def _init():
        out_ref[...] = jnp.zeros_like(out_ref)
        cnt_ref[...] = jnp.zeros_like(cnt_ref)

    a = ap_ref[0] + ap_ref[1]
    wn = wn_ref[...]
    z = (
        t1_ref[...]
        + jnp.dot(gp_ref[0], wn[:HH], preferred_element_type=jnp.float32)
        + jnp.dot(gp_ref[1], wn[HH:], preferred_element_type=jnp.float32)
        + jnp.dot(a, we_ref[...], preferred_element_type=jnp.float32)
        + bn_ref[...]
    )
    y = jnp.maximum(jnp.dot(z, wu_ref[...], preferred_element_type=jnp.float32)
                    + bu_ref[...], 0.0)

    seg = batch_ref[0]
    onehot = (seg == lax.broadcasted_iota(jnp.int32, (N_SUB, R), 0)
              ).astype(jnp.float32)
    out_ref[...] += jnp.dot(onehot, y, preferred_element_type=jnp.float32)
    cnt_ref[...] += jnp.sum(onehot, axis=1, keepdims=True)

    @pl.when(i == NB - 1)
    def _fin():
        out_ref[...] = out_ref[...] / jnp.maximum(cnt_ref[...], 1.0)


def kernel(x, edge_attr, W_node, b_node, W_edge, b_edge, W_upd, b_upd,
           batch, subgraph_idx_batch, edge_index):
    src = edge_index[0]
    dst = edge_index[1]
    we4 = jnp.concatenate([W_edge, (b_node + b_edge)[None, :]], axis=0)
    zg = jnp.zeros((NPAD, HH), jnp.float32)
    za = jnp.zeros((NPAD,), jnp.float32)
    ones_e = jnp.ones((CH,), jnp.float32)
    xs = jnp.stack([x[:, :HH], x[:, HH:]])

    gp, ap = _sc_segment_sums(
        xs, src, dst,
        edge_attr[:, 0], edge_attr[:, 1], edge_attr[:, 2],
        ones_e, zg, za)

    t1 = pl.pallas_call(
        _tc_xw_body,
        grid=(NB,),
        in_specs=[
            pl.BlockSpec((R, H), lambda i: (i, 0)),
            pl.BlockSpec((H, H), lambda i: (0, 0)),
        ],
        out_specs=pl.BlockSpec((R, H), lambda i: (i, 0)),
        out_shape=jax.ShapeDtypeStruct((N_NODES, H), jnp.float32),
    )(x, W_node)

    out = pl.pallas_call(
        _tc_body,
        grid=(NB,),
        in_specs=[
            pl.BlockSpec((R, H), lambda i: (i, 0)),
            pl.BlockSpec((2, R, HH), lambda i: (0, i, 0)),
            pl.BlockSpec((2, R, AW), lambda i: (0, i, 0)),
            pl.BlockSpec((1, 1, R), lambda i: (i, 0, 0)),
            pl.BlockSpec((H, H), lambda i: (0, 0)),
            pl.BlockSpec((AW, H), lambda i: (0, 0)),
            pl.BlockSpec((H, H), lambda i: (0, 0)),
            pl.BlockSpec((1, H), lambda i: (0, 0)),
            pl.BlockSpec((1, H), lambda i: (0, 0)),
        ],
        out_specs=pl.BlockSpec((N_SUB, H), lambda i: (0, 0)),
        out_shape=jax.ShapeDtypeStruct((N_SUB, H), jnp.float32),
        scratch_shapes=[pltpu.VMEM((N_SUB, H), jnp.float32)],
    )(t1, gp, jnp.swapaxes(ap, 1, 2), batch.reshape(NB, 1, R), W_node,
      we4, W_upd,
      b_node[None, :], b_upd[None, :])
    return out


# speedup vs baseline: 1.6180x; 1.0275x over previous
import functools

import jax
import jax.numpy as jnp
from jax import lax
from jax.experimental import pallas as pl
from jax.experimental.pallas import tpu as pltpu
from jax.experimental.pallas import tpu_sc as plsc

N_NODES = 10000
NPAD = 10240
E = 320000
N_SUB = 512
H = 128
AW = 4

HH = H // 2

CH = 80
NCHUNK = E // CH
CPH = NCHUNK // 2 // 16
TILE_ROWS = NPAD // 16

R = 1000
NB = N_NODES // R


def _sc_segment_sums(xs, src, dst, e0, e1, e2, ones_e, zg, za):
    mesh = plsc.VectorSubcoreMesh(core_axis_name="c", subcore_axis_name="s")
    HCH = CPH * CH

    @functools.partial(
        pl.kernel,
        out_type=[
            jax.ShapeDtypeStruct((NPAD, H), jnp.float32),
            jax.ShapeDtypeStruct((2, AW, NPAD), jnp.float32),
        ],
        mesh=mesh,
        compiler_params=pltpu.CompilerParams(use_tc_tiling_on_sc=False),
        scratch_types=[
            pltpu.VMEM((CPH, CH), jnp.int32),
            pltpu.VMEM((CPH, CH), jnp.int32),
            pltpu.VMEM((CH, HH), jnp.float32),
            pltpu.VMEM((CH, HH), jnp.float32),
            pltpu.VMEM((HCH,), jnp.float32),
            pltpu.VMEM((HCH,), jnp.float32),
            pltpu.VMEM((HCH,), jnp.float32),
            pltpu.VMEM((CH,), jnp.float32),
            pltpu.VMEM((TILE_ROWS // 4, HH), jnp.float32),
            pltpu.VMEM((TILE_ROWS,), jnp.float32),
            pltpu.VMEM_SHARED((NPAD, HH), jnp.float32),
            pltpu.VMEM_SHARED((AW, NPAD), jnp.float32),
            pltpu.SemaphoreType.DMA,
            pltpu.SemaphoreType.DMA,
            pltpu.SemaphoreType.DMA,
        ],
    )
    def sc_kernel(xs_hbm, src_hbm, dst_hbm, e0_hbm, e1_hbm, e2_hbm,
                  ones_hbm, zg_hbm, za_hbm, g_out, a_out,
                  srcblk, dstblk, rowb0, rowb1, ec0, ec1, ec2, onesb,
                  stg_g, stg_a, gacc, aacc, sem0, sem1, sem2):
        cid = lax.axis_index("c")
        sid = lax.axis_index("s")
        r0 = sid * TILE_ROWS
        rb = (rowb0, rowb1)
        sems = (sem0, sem1)

        QR = TILE_ROWS // 4
        for q in range(4):
            pltpu.sync_copy(zg_hbm.at[pl.ds(r0 + q * QR, QR)], stg_g)
            pltpu.sync_copy(stg_g, gacc.at[pl.ds(r0 + q * QR, QR)])
        pltpu.sync_copy(za_hbm.at[pl.ds(r0, TILE_ROWS)], stg_a)
        for c in range(AW):
            pltpu.sync_copy(stg_a, aacc.at[c].at[pl.ds(r0, TILE_ROWS)])
        pltpu.sync_copy(ones_hbm, onesb)
        plsc.subcore_barrier()

        own0 = cid * (NCHUNK // 2) + sid * CPH
        oth0 = (1 - cid) * (NCHUNK // 2) + sid * CPH

        def run_half(c0, own):
            pltpu.sync_copy(src_hbm.at[pl.ds(c0, CPH)], srcblk)
            pltpu.sync_copy(dst_hbm.at[pl.ds(c0, CPH)], dstblk)
            if own:
                pltpu.sync_copy(e0_hbm.at[pl.ds(c0 * CH, HCH)], ec0)
                pltpu.sync_copy(e1_hbm.at[pl.ds(c0 * CH, HCH)], ec1)
                pltpu.sync_copy(e2_hbm.at[pl.ds(c0 * CH, HCH)], ec2)
            xh = xs_hbm.at[cid]

            def fire(i, p):
                pltpu.async_copy(xh.at[srcblk.at[i]], rb[p], sems[p])

            def drain(i, p):
                pltpu.make_async_copy(
                    xh.at[srcblk.at[i]], rb[p], sems[p]).wait()
                if own:
                    dsts = dstblk.at[i]
                    cs0 = pltpu.async_copy(ec0.at[pl.ds(i * CH, CH)],
                                           aacc.at[0].at[dsts], sem2,
                                           add=True)
                    cs1 = pltpu.async_copy(ec1.at[pl.ds(i * CH, CH)],
                                           aacc.at[1].at[dsts], sem2,
                                           add=True)
                    cs2 = pltpu.async_copy(ec2.at[pl.ds(i * CH, CH)],
                                           aacc.at[2].at[dsts], sem2,
                                           add=True)
                    cs3 = pltpu.async_copy(onesb, aacc.at[3].at[dsts],
                                           sem2, add=True)
                    pltpu.sync_copy(rb[p], gacc.at[dstblk.at[i]], add=True)
                    cs0.wait()
                    cs1.wait()
                    cs2.wait()
                    cs3.wait()
                else:
                    pltpu.sync_copy(rb[p], gacc.at[dstblk.at[i]], add=True)

            fire(0, 0)

            def pair(k, carry):
                i = 2 * k
                fire(i + 1, 1)
                drain(i, 0)
                fire(i + 2, 0)
                drain(i + 1, 1)
                return carry

            lax.fori_loop(0, (CPH - 1) // 2, pair, 0)
            drain(CPH - 1, 0)

        run_half(own0, True)
        run_half(oth0, False)
        plsc.subcore_barrier()

        for q in range(4):
            pltpu.sync_copy(gacc.at[pl.ds(r0 + q * QR, QR)], stg_g)
            pltpu.sync_copy(
                stg_g,
                g_out.at[pl.ds(r0 + q * QR, QR), pl.ds(cid * HH, HH)])
        for c in range(AW):
            pltpu.sync_copy(aacc.at[c].at[pl.ds(r0, TILE_ROWS)], stg_a)
            pltpu.sync_copy(
                stg_a, a_out.at[cid].at[c].at[pl.ds(r0, TILE_ROWS)])

    gp, ap = sc_kernel(
        xs,
        src.reshape(NCHUNK, CH),
        dst.reshape(NCHUNK, CH),
        e0, e1, e2, ones_e, zg, za)
    return gp, ap


def _tc_xw_body(x_ref, wn_ref, t_ref):
    t_ref[...] = jnp.dot(x_ref[...], wn_ref[...],
                         preferred_element_type=jnp.float32)


def _tc_body(t1_ref, gp_ref, ap_ref, batch_ref, wn_ref, we_ref, wu_ref,
             bn_ref, bu_ref, out_ref, cnt_ref):
    i = pl.program_id(0)

    @pl.when(i == 0)
    def _init():
        out_ref[...] = jnp.zeros_like(out_ref)
        cnt_ref[...] = jnp.zeros_like(cnt_ref)

    a = ap_ref[0] + ap_ref[1]
    z = (
        t1_ref[...]
        + jnp.dot(gp_ref[...], wn_ref[...], preferred_element_type=jnp.float32)
        + jnp.dot(a, we_ref[...], preferred_element_type=jnp.float32)
        + bn_ref[...]
    )
    y = jnp.maximum(jnp.dot(z, wu_ref[...], preferred_element_type=jnp.float32)
                    + bu_ref[...], 0.0)

    seg = batch_ref[0]
    onehot = (seg == lax.broadcasted_iota(jnp.int32, (N_SUB, R), 0)
              ).astype(jnp.float32)
    out_ref[...] += jnp.dot(onehot, y, preferred_element_type=jnp.float32)
    cnt_ref[...] += jnp.sum(onehot, axis=1, keepdims=True)

    @pl.when(i == NB - 1)
    def _fin():
        out_ref[...] = out_ref[...] / jnp.maximum(cnt_ref[...], 1.0)


def kernel(x, edge_attr, W_node, b_node, W_edge, b_edge, W_upd, b_upd,
           batch, subgraph_idx_batch, edge_index):
    src = edge_index[0]
    dst = edge_index[1]
    we4 = jnp.concatenate([W_edge, (b_node + b_edge)[None, :]], axis=0)
    zg = jnp.zeros((NPAD, HH), jnp.float32)
    za = jnp.zeros((NPAD,), jnp.float32)
    ones_e = jnp.ones((CH,), jnp.float32)
    xs = jnp.stack([x[:, :HH], x[:, HH:]])

    gp, ap = _sc_segment_sums(
        xs, src, dst,
        edge_attr[:, 0], edge_attr[:, 1], edge_attr[:, 2],
        ones_e, zg, za)

    t1 = pl.pallas_call(
        _tc_xw_body,
        grid=(NB,),
        in_specs=[
            pl.BlockSpec((R, H), lambda i: (i, 0)),
            pl.BlockSpec((H, H), lambda i: (0, 0)),
        ],
        out_specs=pl.BlockSpec((R, H), lambda i: (i, 0)),
        out_shape=jax.ShapeDtypeStruct((N_NODES, H), jnp.float32),
    )(x, W_node)

    out = pl.pallas_call(
        _tc_body,
        grid=(NB,),
        in_specs=[
            pl.BlockSpec((R, H), lambda i: (i, 0)),
            pl.BlockSpec((R, H), lambda i: (i, 0)),
            pl.BlockSpec((2, R, AW), lambda i: (0, i, 0)),
            pl.BlockSpec((1, 1, R), lambda i: (i, 0, 0)),
            pl.BlockSpec((H, H), lambda i: (0, 0)),
            pl.BlockSpec((AW, H), lambda i: (0, 0)),
            pl.BlockSpec((H, H), lambda i: (0, 0)),
            pl.BlockSpec((1, H), lambda i: (0, 0)),
            pl.BlockSpec((1, H), lambda i: (0, 0)),
        ],
        out_specs=pl.BlockSpec((N_SUB, H), lambda i: (0, 0)),
        out_shape=jax.ShapeDtypeStruct((N_SUB, H), jnp.float32),
        scratch_shapes=[pltpu.VMEM((N_SUB, H), jnp.float32)],
    )(t1, gp, jnp.swapaxes(ap, 1, 2), batch.reshape(NB, 1, R), W_node,
      we4, W_upd,
      b_node[None, :], b_upd[None, :])
    return out


# speedup vs baseline: 2.0171x; 1.2466x over previous
import functools

import jax
import jax.numpy as jnp
from jax import lax
from jax.experimental import pallas as pl
from jax.experimental.pallas import tpu as pltpu
from jax.experimental.pallas import tpu_sc as plsc

N_NODES = 10000
NPAD = 10240
E = 320000
N_SUB = 512
H = 128
AW = 4

HH = H // 2

CH = 80
NCHUNK = E // CH
CPH = NCHUNK // 2 // 16
TILE_ROWS = NPAD // 16

R = 1000
NB = N_NODES // R


def _sc_segment_sums(xs, srcdst, e0, e1, e2, ones_e, zg, za):
    mesh = plsc.VectorSubcoreMesh(core_axis_name="c", subcore_axis_name="s")
    HCH = CPH * CH

    @functools.partial(
        pl.kernel,
        out_type=[
            jax.ShapeDtypeStruct((NPAD, H), jnp.float32),
            jax.ShapeDtypeStruct((2, AW, NPAD), jnp.float32),
        ],
        mesh=mesh,
        compiler_params=pltpu.CompilerParams(use_tc_tiling_on_sc=False),
        scratch_types=[
            pltpu.VMEM((CPH, CH), jnp.int32),
            pltpu.VMEM((CPH, CH), jnp.int32),
            pltpu.VMEM((CH, HH), jnp.float32),
            pltpu.VMEM((CH, HH), jnp.float32),
            pltpu.VMEM((CH, HH), jnp.float32),
            pltpu.VMEM((HCH,), jnp.float32),
            pltpu.VMEM((HCH,), jnp.float32),
            pltpu.VMEM((HCH,), jnp.float32),
            pltpu.VMEM((CH,), jnp.float32),
            pltpu.VMEM((TILE_ROWS // 4, HH), jnp.float32),
            pltpu.VMEM((TILE_ROWS,), jnp.float32),
            pltpu.VMEM_SHARED((NPAD, HH), jnp.float32),
            pltpu.VMEM_SHARED((AW, NPAD), jnp.float32),
            pltpu.SemaphoreType.DMA,
            pltpu.SemaphoreType.DMA,
            pltpu.SemaphoreType.DMA,
            pltpu.SemaphoreType.DMA,
        ],
    )
    def sc_kernel(xs_hbm, srcdst_hbm, e0_hbm, e1_hbm, e2_hbm,
                  ones_hbm, zg_hbm, za_hbm, g_out, a_out,
                  srcblk, dstblk, rowb0, rowb1, rowb2, ec0, ec1, ec2, onesb,
                  stg_g, stg_a, gacc, aacc, sem0, sem1, sem3, sem2):
        cid = lax.axis_index("c")
        sid = lax.axis_index("s")
        r0 = sid * TILE_ROWS
        rb = (rowb0, rowb1, rowb2)
        sems = (sem0, sem1, sem3)

        QR = TILE_ROWS // 4
        for q in range(4):
            pltpu.sync_copy(zg_hbm.at[pl.ds(r0 + q * QR, QR)], stg_g)
            pltpu.sync_copy(stg_g, gacc.at[pl.ds(r0 + q * QR, QR)])
        pltpu.sync_copy(za_hbm.at[pl.ds(r0, TILE_ROWS)], stg_a)
        for c in range(AW):
            pltpu.sync_copy(stg_a, aacc.at[c].at[pl.ds(r0, TILE_ROWS)])
        pltpu.sync_copy(ones_hbm, onesb)
        plsc.subcore_barrier()

        own0 = cid * (NCHUNK // 2) + sid * CPH
        oth0 = (1 - cid) * (NCHUNK // 2) + sid * CPH

        def run_half(c0, own):
            pltpu.sync_copy(srcdst_hbm.at[0].at[pl.ds(c0, CPH)], srcblk)
            pltpu.sync_copy(srcdst_hbm.at[1].at[pl.ds(c0, CPH)], dstblk)
            if own:
                pltpu.sync_copy(e0_hbm.at[pl.ds(c0 * CH, HCH)], ec0)
                pltpu.sync_copy(e1_hbm.at[pl.ds(c0 * CH, HCH)], ec1)
                pltpu.sync_copy(e2_hbm.at[pl.ds(c0 * CH, HCH)], ec2)
            xh = xs_hbm.at[cid]

            def fire(i, p):
                pltpu.async_copy(xh.at[srcblk.at[i]], rb[p], sems[p])

            def drain(i, p):
                pltpu.make_async_copy(
                    xh.at[srcblk.at[i]], rb[p], sems[p]).wait()
                if own:
                    dsts = dstblk.at[i]
                    cs0 = pltpu.async_copy(ec0.at[pl.ds(i * CH, CH)],
                                           aacc.at[0].at[dsts], sem2,
                                           add=True)
                    cs1 = pltpu.async_copy(ec1.at[pl.ds(i * CH, CH)],
                                           aacc.at[1].at[dsts], sem2,
                                           add=True)
                    cs2 = pltpu.async_copy(ec2.at[pl.ds(i * CH, CH)],
                                           aacc.at[2].at[dsts], sem2,
                                           add=True)
                    cs3 = pltpu.async_copy(onesb, aacc.at[3].at[dsts],
                                           sem2, add=True)
                    pltpu.sync_copy(rb[p], gacc.at[dstblk.at[i]], add=True)
                    cs0.wait()
                    cs1.wait()
                    cs2.wait()
                    cs3.wait()
                else:
                    pltpu.sync_copy(rb[p], gacc.at[dstblk.at[i]], add=True)

            fire(0, 0)
            fire(1, 1)

            def triple(k, carry):
                i = 3 * k
                fire(i + 2, 2)
                drain(i, 0)
                fire(i + 3, 0)
                drain(i + 1, 1)
                fire(i + 4, 1)
                drain(i + 2, 2)
                return carry

            lax.fori_loop(0, (CPH - 2) // 3, triple, 0)
            drain(CPH - 2, 0)
            drain(CPH - 1, 1)

        run_half(own0, True)
        run_half(oth0, False)
        plsc.subcore_barrier()

        for q in range(4):
            pltpu.sync_copy(gacc.at[pl.ds(r0 + q * QR, QR)], stg_g)
            pltpu.sync_copy(
                stg_g,
                g_out.at[pl.ds(r0 + q * QR, QR), pl.ds(cid * HH, HH)])
        for c in range(AW):
            pltpu.sync_copy(aacc.at[c].at[pl.ds(r0, TILE_ROWS)], stg_a)
            pltpu.sync_copy(
                stg_a, a_out.at[cid].at[c].at[pl.ds(r0, TILE_ROWS)])

    gp, ap = sc_kernel(xs, srcdst, e0, e1, e2, ones_e, zg, za)
    return gp, ap


def _tc_xw_body(x_ref, wn_ref, t_ref):
    t_ref[...] = jnp.dot(x_ref[...], wn_ref[...],
                         preferred_element_type=jnp.float32)


def _tc_body(t1_ref, gp_ref, ap_ref, batch_ref, wn_ref, we_ref, wu_ref,
             bn_ref, bu_ref, out_ref, cnt_ref):
    i = pl.program_id(0)

    @pl.when(i == 0)
    def _init():
        out_ref[...] = jnp.zeros_like(out_ref)
        cnt_ref[...] = jnp.zeros_like(cnt_ref)

    a = ap_ref[0] + ap_ref[1]
    z = (
        t1_ref[...]
        + jnp.dot(gp_ref[...], wn_ref[...], preferred_element_type=jnp.float32)
        + jnp.dot(a, we_ref[...], preferred_element_type=jnp.float32)
        + bn_ref[...]
    )
    y = jnp.maximum(jnp.dot(z, wu_ref[...], preferred_element_type=jnp.float32)
                    + bu_ref[...], 0.0)

    seg = batch_ref[0]
    onehot = (seg == lax.broadcasted_iota(jnp.int32, (N_SUB, R), 0)
              ).astype(jnp.float32)
    out_ref[...] += jnp.dot(onehot, y, preferred_element_type=jnp.float32)
    cnt_ref[...] += jnp.sum(onehot, axis=1, keepdims=True)

    @pl.when(i == NB - 1)
    def _fin():
        out_ref[...] = out_ref[...] / jnp.maximum(cnt_ref[...], 1.0)


def kernel(x, edge_attr, W_node, b_node, W_edge, b_edge, W_upd, b_upd,
           batch, subgraph_idx_batch, edge_index):
    we4 = jnp.concatenate([W_edge, (b_node + b_edge)[None, :]], axis=0)
    zg = jnp.zeros((NPAD, HH), jnp.float32)
    za = jnp.zeros((NPAD,), jnp.float32)
    ones_e = jnp.ones((CH,), jnp.float32)
    xs = jnp.stack([x[:, :HH], x[:, HH:]])

    gp, ap = _sc_segment_sums(
        xs, edge_index.reshape(2, NCHUNK, CH),
        edge_attr[:, 0], edge_attr[:, 1], edge_attr[:, 2],
        ones_e, zg, za)

    t1 = pl.pallas_call(
        _tc_xw_body,
        grid=(NB,),
        in_specs=[
            pl.BlockSpec((R, H), lambda i: (i, 0)),
            pl.BlockSpec((H, H), lambda i: (0, 0)),
        ],
        out_specs=pl.BlockSpec((R, H), lambda i: (i, 0)),
        out_shape=jax.ShapeDtypeStruct((N_NODES, H), jnp.float32),
    )(x, W_node)

    out = pl.pallas_call(
        _tc_body,
        grid=(NB,),
        in_specs=[
            pl.BlockSpec((R, H), lambda i: (i, 0)),
            pl.BlockSpec((R, H), lambda i: (i, 0)),
            pl.BlockSpec((2, R, AW), lambda i: (0, i, 0)),
            pl.BlockSpec((1, 1, R), lambda i: (i, 0, 0)),
            pl.BlockSpec((H, H), lambda i: (0, 0)),
            pl.BlockSpec((AW, H), lambda i: (0, 0)),
            pl.BlockSpec((H, H), lambda i: (0, 0)),
            pl.BlockSpec((1, H), lambda i: (0, 0)),
            pl.BlockSpec((1, H), lambda i: (0, 0)),
        ],
        out_specs=pl.BlockSpec((N_SUB, H), lambda i: (0, 0)),
        out_shape=jax.ShapeDtypeStruct((N_SUB, H), jnp.float32),
        scratch_shapes=[pltpu.VMEM((N_SUB, H), jnp.float32)],
    )(t1, gp, jnp.swapaxes(ap, 1, 2), batch.reshape(NB, 1, R), W_node,
      we4, W_upd,
      b_node[None, :], b_upd[None, :])
    return out


# speedup vs baseline: 2.1935x; 1.0875x over previous
import functools

import jax
import jax.numpy as jnp
from jax import lax
from jax.experimental import pallas as pl
from jax.experimental.pallas import tpu as pltpu
from jax.experimental.pallas import tpu_sc as plsc

N_NODES = 10000
NPAD = 10240
E = 320000
N_SUB = 512
H = 128
AW = 4

HH = H // 2

CH = 80
NCHUNK = E // CH
CPH = NCHUNK // 2 // 16
TILE_ROWS = NPAD // 16

R = 1000
NB = N_NODES // R


def _sc_segment_sums(xs, srcdst, e0, e1, e2, ones_e, zg, za):
    mesh = plsc.VectorSubcoreMesh(core_axis_name="c", subcore_axis_name="s")
    HCH = CPH * CH

    @functools.partial(
        pl.kernel,
        out_type=[
            jax.ShapeDtypeStruct((NPAD, H), jnp.float32),
            jax.ShapeDtypeStruct((2, AW, NPAD), jnp.float32),
        ],
        mesh=mesh,
        compiler_params=pltpu.CompilerParams(use_tc_tiling_on_sc=False),
        scratch_types=[
            pltpu.VMEM((CPH, CH), jnp.int32),
            pltpu.VMEM((CPH, CH), jnp.int32),
            pltpu.VMEM((CH, HH), jnp.float32),
            pltpu.VMEM((CH, HH), jnp.float32),
            pltpu.VMEM((CH, HH), jnp.float32),
            pltpu.VMEM((CH, HH), jnp.float32),
            pltpu.VMEM((CH, HH), jnp.float32),
            pltpu.VMEM((HCH,), jnp.float32),
            pltpu.VMEM((HCH,), jnp.float32),
            pltpu.VMEM((HCH,), jnp.float32),
            pltpu.VMEM((CH,), jnp.float32),
            pltpu.VMEM((TILE_ROWS // 4, HH), jnp.float32),
            pltpu.VMEM((TILE_ROWS,), jnp.float32),
            pltpu.VMEM_SHARED((NPAD, HH), jnp.float32),
            pltpu.VMEM_SHARED((AW, NPAD), jnp.float32),
            pltpu.SemaphoreType.DMA,
            pltpu.SemaphoreType.DMA,
            pltpu.SemaphoreType.DMA,
            pltpu.SemaphoreType.DMA,
            pltpu.SemaphoreType.DMA,
            pltpu.SemaphoreType.DMA,
        ],
    )
    def sc_kernel(xs_hbm, srcdst_hbm, e0_hbm, e1_hbm, e2_hbm,
                  ones_hbm, zg_hbm, za_hbm, g_out, a_out,
                  srcblk, dstblk, rowb0, rowb1, rowb2, rowb3, rowb4,
                  ec0, ec1, ec2, onesb,
                  stg_g, stg_a, gacc, aacc,
                  sem0, sem1, sem3, sem4, sem5, sem2):
        cid = lax.axis_index("c")
        sid = lax.axis_index("s")
        r0 = sid * TILE_ROWS
        rb = (rowb0, rowb1, rowb2, rowb3, rowb4)
        sems = (sem0, sem1, sem3, sem4, sem5)

        QR = TILE_ROWS // 4
        for q in range(4):
            pltpu.sync_copy(zg_hbm.at[pl.ds(r0 + q * QR, QR)], stg_g)
            pltpu.sync_copy(stg_g, gacc.at[pl.ds(r0 + q * QR, QR)])
        pltpu.sync_copy(za_hbm.at[pl.ds(r0, TILE_ROWS)], stg_a)
        for c in range(AW):
            pltpu.sync_copy(stg_a, aacc.at[c].at[pl.ds(r0, TILE_ROWS)])
        pltpu.sync_copy(ones_hbm, onesb)
        plsc.subcore_barrier()

        own0 = cid * (NCHUNK // 2) + sid * CPH
        oth0 = (1 - cid) * (NCHUNK // 2) + sid * CPH

        def run_half(c0, own):
            pltpu.sync_copy(srcdst_hbm.at[0].at[pl.ds(c0, CPH)], srcblk)
            pltpu.sync_copy(srcdst_hbm.at[1].at[pl.ds(c0, CPH)], dstblk)
            if own:
                pltpu.sync_copy(e0_hbm.at[pl.ds(c0 * CH, HCH)], ec0)
                pltpu.sync_copy(e1_hbm.at[pl.ds(c0 * CH, HCH)], ec1)
                pltpu.sync_copy(e2_hbm.at[pl.ds(c0 * CH, HCH)], ec2)
            xh = xs_hbm.at[cid]

            def fire(i, p):
                pltpu.async_copy(xh.at[srcblk.at[i]], rb[p], sems[p])

            def drain(i, p):
                pltpu.make_async_copy(
                    xh.at[srcblk.at[i]], rb[p], sems[p]).wait()
                if own:
                    dsts = dstblk.at[i]
                    cs0 = pltpu.async_copy(ec0.at[pl.ds(i * CH, CH)],
                                           aacc.at[0].at[dsts], sem2,
                                           add=True)
                    cs1 = pltpu.async_copy(ec1.at[pl.ds(i * CH, CH)],
                                           aacc.at[1].at[dsts], sem2,
                                           add=True)
                    cs2 = pltpu.async_copy(ec2.at[pl.ds(i * CH, CH)],
                                           aacc.at[2].at[dsts], sem2,
                                           add=True)
                    cs3 = pltpu.async_copy(onesb, aacc.at[3].at[dsts],
                                           sem2, add=True)
                    pltpu.sync_copy(rb[p], gacc.at[dstblk.at[i]], add=True)
                    cs0.wait()
                    cs1.wait()
                    cs2.wait()
                    cs3.wait()
                else:
                    pltpu.sync_copy(rb[p], gacc.at[dstblk.at[i]], add=True)

            NBUF = 5
            for b in range(NBUF):
                fire(b, b)

            def ring(k, carry):
                i = NBUF * k
                for b in range(NBUF):
                    drain(i + b, b)
                    fire(i + NBUF + b, b)
                return carry

            lax.fori_loop(0, CPH // NBUF - 1, ring, 0)
            for b in range(NBUF):
                drain(CPH - NBUF + b, b)

        run_half(own0, True)
        run_half(oth0, False)
        plsc.subcore_barrier()

        for q in range(4):
            pltpu.sync_copy(gacc.at[pl.ds(r0 + q * QR, QR)], stg_g)
            pltpu.sync_copy(
                stg_g,
                g_out.at[pl.ds(r0 + q * QR, QR), pl.ds(cid * HH, HH)])
        for c in range(AW):
            pltpu.sync_copy(aacc.at[c].at[pl.ds(r0, TILE_ROWS)], stg_a)
            pltpu.sync_copy(
                stg_a, a_out.at[cid].at[c].at[pl.ds(r0, TILE_ROWS)])

    gp, ap = sc_kernel(xs, srcdst, e0, e1, e2, ones_e, zg, za)
    return gp, ap


def _tc_xw_body(x_ref, wn_ref, t_ref):
    t_ref[...] = jnp.dot(x_ref[...], wn_ref[...],
                         preferred_element_type=jnp.float32)


def _tc_body(t1_ref, gp_ref, ap_ref, batch_ref, wn_ref, we_ref, wu_ref,
             bn_ref, bu_ref, out_ref, cnt_ref):
    i = pl.program_id(0)

    @pl.when(i == 0)
    def _init():
        out_ref[...] = jnp.zeros_like(out_ref)
        cnt_ref[...] = jnp.zeros_like(cnt_ref)

    a = ap_ref[0] + ap_ref[1]
    z = (
        t1_ref[...]
        + jnp.dot(gp_ref[...], wn_ref[...], preferred_element_type=jnp.float32)
        + jnp.dot(a, we_ref[...], preferred_element_type=jnp.float32)
        + bn_ref[...]
    )
    y = jnp.maximum(jnp.dot(z, wu_ref[...], preferred_element_type=jnp.float32)
                    + bu_ref[...], 0.0)

    seg = batch_ref[0]
    onehot = (seg == lax.broadcasted_iota(jnp.int32, (N_SUB, R), 0)
              ).astype(jnp.float32)
    out_ref[...] += jnp.dot(onehot, y, preferred_element_type=jnp.float32)
    cnt_ref[...] += jnp.sum(onehot, axis=1, keepdims=True)

    @pl.when(i == NB - 1)
    def _fin():
        out_ref[...] = out_ref[...] / jnp.maximum(cnt_ref[...], 1.0)


def kernel(x, edge_attr, W_node, b_node, W_edge, b_edge, W_upd, b_upd,
           batch, subgraph_idx_batch, edge_index):
    we4 = jnp.concatenate([W_edge, (b_node + b_edge)[None, :]], axis=0)
    zg = jnp.zeros((NPAD, HH), jnp.float32)
    za = jnp.zeros((NPAD,), jnp.float32)
    ones_e = jnp.ones((CH,), jnp.float32)
    xs = jnp.stack([x[:, :HH], x[:, HH:]])

    gp, ap = _sc_segment_sums(
        xs, edge_index.reshape(2, NCHUNK, CH),
        edge_attr[:, 0], edge_attr[:, 1], edge_attr[:, 2],
        ones_e, zg, za)

    t1 = pl.pallas_call(
        _tc_xw_body,
        grid=(NB,),
        in_specs=[
            pl.BlockSpec((R, H), lambda i: (i, 0)),
            pl.BlockSpec((H, H), lambda i: (0, 0)),
        ],
        out_specs=pl.BlockSpec((R, H), lambda i: (i, 0)),
        out_shape=jax.ShapeDtypeStruct((N_NODES, H), jnp.float32),
    )(x, W_node)

    out = pl.pallas_call(
        _tc_body,
        grid=(NB,),
        in_specs=[
            pl.BlockSpec((R, H), lambda i: (i, 0)),
            pl.BlockSpec((R, H), lambda i: (i, 0)),
            pl.BlockSpec((2, R, AW), lambda i: (0, i, 0)),
            pl.BlockSpec((1, 1, R), lambda i: (i, 0, 0)),
            pl.BlockSpec((H, H), lambda i: (0, 0)),
            pl.BlockSpec((AW, H), lambda i: (0, 0)),
            pl.BlockSpec((H, H), lambda i: (0, 0)),
            pl.BlockSpec((1, H), lambda i: (0, 0)),
            pl.BlockSpec((1, H), lambda i: (0, 0)),
        ],
        out_specs=pl.BlockSpec((N_SUB, H), lambda i: (0, 0)),
        out_shape=jax.ShapeDtypeStruct((N_SUB, H), jnp.float32),
        scratch_shapes=[pltpu.VMEM((N_SUB, H), jnp.float32)],
    )(t1, gp, jnp.swapaxes(ap, 1, 2), batch.reshape(NB, 1, R), W_node,
      we4, W_upd,
      b_node[None, :], b_upd[None, :])
    return out
